# Initial kernel scaffold; baseline (speedup 1.0000x reference)
import jax
import jax.numpy as jnp
from jax.experimental import pallas as pl


def kernel(x, table):
    raise NotImplementedError("write your pallas kernel here")


# speedup vs baseline: 1.4071x; 1.4071x over previous
import functools

import jax
import jax.numpy as jnp
from jax import lax
from jax.experimental import pallas as pl
from jax.experimental.pallas import tpu as pltpu
from jax.experimental.pallas import tpu_sc as plsc

VOCAB = 99996
DIM = 64
BATCH = 4096
FIELDS = 26

NC = 2
NS = 16
NW = NC * NS
B_PER_W = BATCH // NW
ROWS_PER_CHUNK = 16
NCHUNK = B_PER_W // ROWS_PER_CHUNK
SUB = 4
IDX_PER_SUB = ROWS_PER_CHUNK * FIELDS // SUB
NVREG = DIM // 16


def _fm_body(x_hbm, table_hbm, out_hbm, idx_v, rows_v, out_v, sem):
    wid = lax.axis_index("s") * NC + lax.axis_index("c")
    pltpu.sync_copy(x_hbm.at[wid], idx_v)
    lane = lax.broadcasted_iota(jnp.int32, (16,), 0)
    perms = [lane ^ sh for sh in (8, 4, 2, 1)]

    def chunk_body(c, carry):
        copies = []
        for s in range(SUB):
            copies.append(
                pltpu.async_copy(
                    table_hbm.at[idx_v.at[c, s]],
                    rows_v.at[pl.ds(s * IDX_PER_SUB, IDX_PER_SUB)],
                    sem,
                )
            )
        for cp in copies:
            cp.wait()

        out_vec = jnp.zeros((16,), jnp.float32)
        for j in range(ROWS_PER_CHUNK):
            acc = [jnp.zeros((16,), jnp.float32) for _ in range(NVREG)]
            accq = jnp.zeros((16,), jnp.float32)
            for f in range(FIELDS):
                r = j * FIELDS + f
                for i in range(NVREG):
                    v = rows_v[r, pl.ds(i * 16, 16)]
                    acc[i] = acc[i] + v
                    accq = accq + v * v
            tot = -accq
            for i in range(NVREG):
                tot = tot + acc[i] * acc[i]
            for p in perms:
                tot = tot + tot.at[p].get(mode="promise_in_bounds")
            out_vec = jnp.where(lane == j, tot, out_vec)
        out_v[c] = out_vec
        return carry

    lax.fori_loop(0, NCHUNK, chunk_body, 0)
    pltpu.sync_copy(out_v, out_hbm.at[wid])


@jax.jit
def kernel(x, table):
    xr = x.astype(jnp.int32).reshape(NW, NCHUNK, SUB, IDX_PER_SUB)
    mesh = plsc.VectorSubcoreMesh(core_axis_name="c", subcore_axis_name="s")
    fm = pl.kernel(
        _fm_body,
        out_type=jax.ShapeDtypeStruct((NW, NCHUNK, 16), jnp.float32),
        mesh=mesh,
        scratch_types=[
            pltpu.VMEM((NCHUNK, SUB, IDX_PER_SUB), jnp.int32),
            pltpu.VMEM((ROWS_PER_CHUNK * FIELDS, DIM), jnp.float32),
            pltpu.VMEM((NCHUNK, 16), jnp.float32),
            pltpu.SemaphoreType.DMA,
        ],
        compiler_params=pltpu.CompilerParams(use_tc_tiling_on_sc=False),
    )
    out = fm(xr, table)
    return out.reshape(BATCH)
